# baseline (speedup 1.0000x reference)
import functools

import jax
import jax.numpy as jnp
from jax import lax
from jax.experimental import pallas as pl
from jax.experimental.pallas import tpu as pltpu
from jax.experimental.pallas import tpu_sc as plsc

_NUM_CORES = 2
_NUM_SUBCORES = 16
_NW = _NUM_CORES * _NUM_SUBCORES

_GATHER_DNUMS = lax.GatherDimensionNumbers(
    offset_dims=(), collapsed_slice_dims=(0,), start_index_map=(0,))


def _shuffle(v, idx):
    return lax.gather(v, idx[:, None], _GATHER_DNUMS, (1,),
                      mode=lax.GatherScatterMode.PROMISE_IN_BOUNDS)


def _build(E, D, C):
    EW = E // _NW
    NCH = EW // C
    mesh = plsc.VectorSubcoreMesh(core_axis_name="c", subcore_axis_name="s")

    @functools.partial(
        pl.kernel,
        mesh=mesh,
        out_type=jax.ShapeDtypeStruct((E,), jnp.float32),
        scratch_types=[
            pltpu.VMEM((C,), jnp.int32),
            pltpu.VMEM((C,), jnp.int32),
            pltpu.VMEM((C,), jnp.int32),
            pltpu.VMEM((C,), jnp.int32),
            pltpu.VMEM((C, D), jnp.float32),
            pltpu.VMEM((C, D), jnp.float32),
            pltpu.VMEM((C, D), jnp.float32),
            pltpu.VMEM((C, D), jnp.float32),
            pltpu.VMEM((C,), jnp.float32),
            pltpu.VMEM((C,), jnp.float32),
            pltpu.SemaphoreType.DMA,
            pltpu.SemaphoreType.DMA,
            pltpu.SemaphoreType.DMA,
            pltpu.SemaphoreType.DMA,
        ],
    )
    def k(x_hbm, s_hbm, d_hbm, out_hbm, sidx0, sidx1, didx0, didx1,
          srows0, srows1, drows0, drows1, outv0, outv1,
          gsem0, gsem1, osem0, osem1):
        sidx = (sidx0, sidx1)
        didx = (didx0, didx1)
        srows = (srows0, srows1)
        drows = (drows0, drows1)
        outv = (outv0, outv1)
        gsems = (gsem0, gsem1)
        osems = (osem0, osem1)
        wid = lax.axis_index("s") * _NUM_CORES + lax.axis_index("c")
        base_w = wid * EW
        lane = lax.iota(jnp.int32, 16)

        def stage_idx(ci, b):
            base = base_w + ci * C
            pltpu.sync_copy(s_hbm.at[pl.ds(base, C)], sidx[b])
            pltpu.sync_copy(d_hbm.at[pl.ds(base, C)], didx[b])

        def gathers(b):
            return (
                pltpu.make_async_copy(x_hbm.at[sidx[b]], srows[b], gsems[b]),
                pltpu.make_async_copy(x_hbm.at[didx[b]], drows[b], gsems[b]),
            )

        def out_copy(ci, b):
            return pltpu.make_async_copy(
                outv[b], out_hbm.at[pl.ds(base_w + ci * C, C)], osems[b])

        def compute(b):
            def group(e0):
                vec = jnp.zeros((16,), jnp.float32)
                for j in range(16):
                    e = e0 + j
                    acc0 = (srows[b][e, pl.ds(0, 16)]
                            * drows[b][e, pl.ds(0, 16)])
                    acc1 = (srows[b][e, pl.ds(16, 16)]
                            * drows[b][e, pl.ds(16, 16)])
                    for kk in range(2, D // 16, 2):
                        acc0 = acc0 + (srows[b][e, pl.ds(kk * 16, 16)]
                                       * drows[b][e, pl.ds(kk * 16, 16)])
                        acc1 = acc1 + (srows[b][e, pl.ds(kk * 16 + 16, 16)]
                                       * drows[b][e, pl.ds(kk * 16 + 16, 16)])
                    acc = acc0 + acc1
                    for dist in (8, 4, 2, 1):
                        acc = acc + _shuffle(acc, lane ^ dist)
                    vec = jnp.where(lane == j, acc, vec)
                outv[b][pl.ds(e0, 16)] = vec

            @plsc.parallel_loop(0, (C // 16) * 16, step=16, unroll=2)
            def _gloop(e0):
                group(e0)
            if C % 16:
                group(C - 16)

        for b in range(2):
            stage_idx(b, b)
            g1, g2 = gathers(b)
            g1.start()
            g2.start()

        def body(i, _):
            for b in range(2):
                ci = 2 * i + b
                g1, g2 = gathers(b)
                g1.wait()
                g2.wait()

                @pl.when(ci + 2 < NCH)
                def _():
                    stage_idx(ci + 2, b)

                @pl.when(ci >= 2)
                def _():
                    out_copy(ci - 2, b).wait()

                compute(b)
                out_copy(ci, b).start()

                @pl.when(ci + 2 < NCH)
                def _():
                    n1, n2 = gathers(b)
                    n1.start()
                    n2.start()
            return 0

        lax.fori_loop(0, NCH // 2, body, 0)
        out_copy(NCH - 2, 0).wait()
        out_copy(NCH - 1, 1).wait()

    return k


def kernel(x, edge_index):
    N, D = x.shape
    E = edge_index.shape[1]
    k = _build(E, D, 200)
    ei = edge_index.astype(jnp.int32)
    return k(x, ei[0], ei[1])


# speedup vs baseline: 1.5179x; 1.5179x over previous
import functools

import jax
import jax.numpy as jnp
from jax import lax
from jax.experimental import pallas as pl
from jax.experimental.pallas import tpu as pltpu
from jax.experimental.pallas import tpu_sc as plsc

_NUM_CORES = 2
_NUM_SUBCORES = 16
_NW = _NUM_CORES * _NUM_SUBCORES

_GATHER_DNUMS = lax.GatherDimensionNumbers(
    offset_dims=(), collapsed_slice_dims=(0,), start_index_map=(0,))


def _shuffle(v, idx):
    return lax.gather(v, idx[:, None], _GATHER_DNUMS, (1,),
                      mode=lax.GatherScatterMode.PROMISE_IN_BOUNDS)


def _build(E, D, C):
    EW = E // _NW
    NCH = EW // C
    mesh = plsc.VectorSubcoreMesh(core_axis_name="c", subcore_axis_name="s")

    @functools.partial(
        pl.kernel,
        mesh=mesh,
        out_type=jax.ShapeDtypeStruct((E,), jnp.float32),
        scratch_types=[
            pltpu.VMEM((C,), jnp.int32),
            pltpu.VMEM((C,), jnp.int32),
            pltpu.VMEM((C,), jnp.int32),
            pltpu.VMEM((C,), jnp.int32),
            pltpu.VMEM((C, D), jnp.float32),
            pltpu.VMEM((C, D), jnp.float32),
            pltpu.VMEM((C, D), jnp.float32),
            pltpu.VMEM((C, D), jnp.float32),
            pltpu.VMEM((C,), jnp.float32),
            pltpu.VMEM((C,), jnp.float32),
            pltpu.SemaphoreType.DMA,
            pltpu.SemaphoreType.DMA,
            pltpu.SemaphoreType.DMA,
            pltpu.SemaphoreType.DMA,
        ],
    )
    def k(x_hbm, s_hbm, d_hbm, out_hbm, sidx0, sidx1, didx0, didx1,
          srows0, srows1, drows0, drows1, outv0, outv1,
          gsem0, gsem1, osem0, osem1):
        sidx = (sidx0, sidx1)
        didx = (didx0, didx1)
        srows = (srows0, srows1)
        drows = (drows0, drows1)
        outv = (outv0, outv1)
        gsems = (gsem0, gsem1)
        osems = (osem0, osem1)
        wid = lax.axis_index("s") * _NUM_CORES + lax.axis_index("c")
        base_w = wid * EW
        lane = lax.iota(jnp.int32, 16)

        def stage_idx(ci, b):
            base = base_w + ci * C
            pltpu.sync_copy(s_hbm.at[pl.ds(base, C)], sidx[b])
            pltpu.sync_copy(d_hbm.at[pl.ds(base, C)], didx[b])

        def gathers(b):
            return (
                pltpu.make_async_copy(x_hbm.at[sidx[b]], srows[b], gsems[b]),
                pltpu.make_async_copy(x_hbm.at[didx[b]], drows[b], gsems[b]),
            )

        def out_copy(ci, b):
            return pltpu.make_async_copy(
                outv[b], out_hbm.at[pl.ds(base_w + ci * C, C)], osems[b])

        def compute(b):
            def group(e0):
                vec = jnp.zeros((16,), jnp.float32)
                for j in range(16):
                    e = e0 + j
                    acc0 = (srows[b][e, pl.ds(0, 16)]
                            * drows[b][e, pl.ds(0, 16)])
                    acc1 = (srows[b][e, pl.ds(16, 16)]
                            * drows[b][e, pl.ds(16, 16)])
                    for kk in range(2, D // 16, 2):
                        acc0 = acc0 + (srows[b][e, pl.ds(kk * 16, 16)]
                                       * drows[b][e, pl.ds(kk * 16, 16)])
                        acc1 = acc1 + (srows[b][e, pl.ds(kk * 16 + 16, 16)]
                                       * drows[b][e, pl.ds(kk * 16 + 16, 16)])
                    acc = acc0 + acc1
                    for dist in (8, 4, 2, 1):
                        acc = acc + _shuffle(acc, lane ^ dist)
                    vec = jnp.where(lane == j, acc, vec)
                outv[b][pl.ds(e0, 16)] = vec

            @plsc.parallel_loop(0, (C // 16) * 16, step=16)
            def _gloop(e0):
                group(e0)
            if C % 16:
                group(C - 16)

        for b in range(2):
            stage_idx(b, b)
            g1, g2 = gathers(b)
            g1.start()
            g2.start()

        def body(i, _):
            for b in range(2):
                ci = 2 * i + b
                g1, g2 = gathers(b)
                g1.wait()
                g2.wait()

                @pl.when(ci + 2 < NCH)
                def _():
                    stage_idx(ci + 2, b)

                @pl.when(ci >= 2)
                def _():
                    out_copy(ci - 2, b).wait()

                compute(b)
                out_copy(ci, b).start()

                @pl.when(ci + 2 < NCH)
                def _():
                    n1, n2 = gathers(b)
                    n1.start()
                    n2.start()
            return 0

        lax.fori_loop(0, NCH // 2, body, 0)
        out_copy(NCH - 2, 0).wait()
        out_copy(NCH - 1, 1).wait()

    return k


def kernel(x, edge_index):
    N, D = x.shape
    E = edge_index.shape[1]
    k = _build(E, D, 200)
    ei = edge_index.astype(jnp.int32)
    return k(x, ei[0], ei[1])


# speedup vs baseline: 1.9141x; 1.2610x over previous
import functools

import jax
import jax.numpy as jnp
from jax import lax
from jax.experimental import pallas as pl
from jax.experimental.pallas import tpu as pltpu
from jax.experimental.pallas import tpu_sc as plsc

_NUM_CORES = 2
_NUM_SUBCORES = 16
_NW = _NUM_CORES * _NUM_SUBCORES

_GATHER_DNUMS = lax.GatherDimensionNumbers(
    offset_dims=(), collapsed_slice_dims=(0,), start_index_map=(0,))


def _shuffle(v, idx):
    return lax.gather(v, idx[:, None], _GATHER_DNUMS, (1,),
                      mode=lax.GatherScatterMode.PROMISE_IN_BOUNDS)


def _build(E, D, C):
    EW = E // _NW
    NCH = EW // C
    mesh = plsc.VectorSubcoreMesh(core_axis_name="c", subcore_axis_name="s")

    @functools.partial(
        pl.kernel,
        mesh=mesh,
        out_type=jax.ShapeDtypeStruct((E,), jnp.float32),
        scratch_types=[
            pltpu.VMEM((C,), jnp.int32),
            pltpu.VMEM((C,), jnp.int32),
            pltpu.VMEM((C,), jnp.int32),
            pltpu.VMEM((C,), jnp.int32),
            pltpu.VMEM((C, D), jnp.float32),
            pltpu.VMEM((C, D), jnp.float32),
            pltpu.VMEM((C, D), jnp.float32),
            pltpu.VMEM((C, D), jnp.float32),
            pltpu.VMEM((C,), jnp.float32),
            pltpu.VMEM((C,), jnp.float32),
            pltpu.SemaphoreType.DMA,
            pltpu.SemaphoreType.DMA,
            pltpu.SemaphoreType.DMA,
            pltpu.SemaphoreType.DMA,
            pltpu.SemaphoreType.DMA,
            pltpu.SemaphoreType.DMA,
        ],
    )
    def k(x_hbm, s_hbm, d_hbm, out_hbm, sidx0, sidx1, didx0, didx1,
          srows0, srows1, drows0, drows1, outv0, outv1,
          gsem0, gsem1, osem0, osem1, isem0, isem1):
        sidx = (sidx0, sidx1)
        didx = (didx0, didx1)
        srows = (srows0, srows1)
        drows = (drows0, drows1)
        outv = (outv0, outv1)
        gsems = (gsem0, gsem1)
        osems = (osem0, osem1)
        isems = (isem0, isem1)
        wid = lax.axis_index("s") * _NUM_CORES + lax.axis_index("c")
        base_w = wid * EW
        lane = lax.iota(jnp.int32, 16)

        def idx_copies(ci, b):
            base = base_w + ci * C
            return (
                pltpu.make_async_copy(
                    s_hbm.at[pl.ds(base, C)], sidx[b], isems[b]),
                pltpu.make_async_copy(
                    d_hbm.at[pl.ds(base, C)], didx[b], isems[b]),
            )

        def gathers(b):
            return (
                pltpu.make_async_copy(x_hbm.at[sidx[b]], srows[b], gsems[b]),
                pltpu.make_async_copy(x_hbm.at[didx[b]], drows[b], gsems[b]),
            )

        def out_copy(ci, b):
            return pltpu.make_async_copy(
                outv[b], out_hbm.at[pl.ds(base_w + ci * C, C)], osems[b])

        def compute(b):
            def group(e0):
                vec = jnp.zeros((16,), jnp.float32)
                for j in range(16):
                    e = e0 + j
                    acc = (srows[b][e, pl.ds(0, 16)]
                           * drows[b][e, pl.ds(0, 16)])
                    for kk in range(1, D // 16):
                        acc = acc + (srows[b][e, pl.ds(kk * 16, 16)]
                                     * drows[b][e, pl.ds(kk * 16, 16)])
                    for dist in (8, 4, 2, 1):
                        acc = acc + _shuffle(acc, lane ^ dist)
                    vec = jnp.where(lane == j, acc, vec)
                outv[b][pl.ds(e0, 16)] = vec

            @plsc.parallel_loop(0, (C // 16) * 16, step=16)
            def _gloop(e0):
                group(e0)
            if C % 16:
                group(C - 16)

        for b in range(2):
            i1, i2 = idx_copies(b, b)
            i1.start()
            i2.start()
            i1.wait()
            i2.wait()
            g1, g2 = gathers(b)
            g1.start()
            g2.start()

        def body(i, _):
            for b in range(2):
                ci = 2 * i + b
                g1, g2 = gathers(b)
                g1.wait()
                g2.wait()

                @pl.when(ci + 2 < NCH)
                def _():
                    n1, n2 = idx_copies(ci + 2, b)
                    n1.start()
                    n2.start()

                @pl.when(ci >= 2)
                def _():
                    out_copy(ci - 2, b).wait()

                compute(b)
                out_copy(ci, b).start()

                @pl.when(ci + 2 < NCH)
                def _():
                    i1, i2 = idx_copies(ci + 2, b)
                    i1.wait()
                    i2.wait()
                    n1, n2 = gathers(b)
                    n1.start()
                    n2.start()
            return 0

        lax.fori_loop(0, NCH // 2, body, 0)
        out_copy(NCH - 2, 0).wait()
        out_copy(NCH - 1, 1).wait()

    return k


def kernel(x, edge_index):
    N, D = x.shape
    E = edge_index.shape[1]
    k = _build(E, D, 200)
    ei = edge_index.astype(jnp.int32)
    return k(x, ei[0], ei[1])
